# baseline (speedup 1.0000x reference)
import functools

import jax
import jax.numpy as jnp
from jax import lax
from jax.experimental import pallas as pl
from jax.experimental.pallas import tpu as pltpu
from jax.experimental.pallas import tpu_sc as plsc

NUM_EXPERTS = 64
TOP_K = 8
MODEL_DIM = 2048
T = 16384
BLOCK_T = 1024

NC = 2
NS = 16
NW = NC * NS
TPW = T // NW
GROUPS = TPW // 16

INT_MIN = -(2 ** 31)


def _tc_block(x_ref, w_ref, probs_ref, keys_ref):
    x = x_ref[...]
    w = w_ref[...]
    logits_t = lax.dot_general(
        w, x,
        dimension_numbers=(((1,), (1,)), ((), ())),
        preferred_element_type=jnp.float32,
    )
    m = jnp.max(logits_t, axis=0, keepdims=True)
    e = jnp.exp(logits_t - m)
    s = jnp.sum(e, axis=0, keepdims=True)
    probs_ref[...] = (e * (1.0 / s)).T
    iota_e = lax.broadcasted_iota(jnp.int32, e.shape, 0)
    keys_ref[...] = (lax.bitcast_convert_type(e, jnp.int32) & ~63) | (63 - iota_e)


def _sc_topk(keys_hbm, tw_hbm, ti_hbm, keys_v, tw_v, ti_v):
    wid = lax.axis_index("s") * NC + lax.axis_index("c")
    base = wid * TPW
    pltpu.sync_copy(keys_hbm.at[:, pl.ds(base, TPW)], keys_v)

    def group(g, _):
        t = [jnp.full((16,), INT_MIN, jnp.int32) for _ in range(TOP_K)]
        for ex in range(NUM_EXPERTS):
            v = keys_v[ex, pl.ds(g * 16, 16)]
            c = [v > tj for tj in t]
            nt = [jnp.where(c[0], v, t[0])]
            for j in range(1, TOP_K):
                nt.append(jnp.where(c[j], jnp.where(c[j - 1], t[j - 1], v), t[j]))
            t = nt
        vals = [lax.bitcast_convert_type((tj & ~63) | 32, jnp.float32) for tj in t]
        ssum = vals[0]
        for vv in vals[1:]:
            ssum = ssum + vv
        inv = 1.0 / ssum
        for j in range(TOP_K):
            tw_v[j, pl.ds(g * 16, 16)] = vals[j] * inv
            ti_v[j, pl.ds(g * 16, 16)] = 63 - (t[j] & 63)
        return 0

    lax.fori_loop(0, GROUPS, group, 0)
    pltpu.sync_copy(tw_v, tw_hbm.at[:, pl.ds(base, TPW)])
    pltpu.sync_copy(ti_v, ti_hbm.at[:, pl.ds(base, TPW)])


@functools.partial(jax.jit, static_argnames=("interpret",))
def _run(hidden_states, weight, interpret=False):
    x = hidden_states.reshape(-1, MODEL_DIM)
    grid = (T // BLOCK_T,)
    probs, keys_t = pl.pallas_call(
        _tc_block,
        grid=grid,
        in_specs=[
            pl.BlockSpec((BLOCK_T, MODEL_DIM), lambda i: (i, 0)),
            pl.BlockSpec((NUM_EXPERTS, MODEL_DIM), lambda i: (0, 0)),
        ],
        out_specs=[
            pl.BlockSpec((BLOCK_T, NUM_EXPERTS), lambda i: (i, 0)),
            pl.BlockSpec((NUM_EXPERTS, BLOCK_T), lambda i: (0, i)),
        ],
        out_shape=[
            jax.ShapeDtypeStruct((T, NUM_EXPERTS), jnp.float32),
            jax.ShapeDtypeStruct((NUM_EXPERTS, T), jnp.int32),
        ],
        interpret=interpret,
    )(x, weight)

    sc_call = pl.kernel(
        _sc_topk,
        out_type=[
            jax.ShapeDtypeStruct((TOP_K, T), jnp.float32),
            jax.ShapeDtypeStruct((TOP_K, T), jnp.int32),
        ],
        mesh=plsc.VectorSubcoreMesh(core_axis_name="c", subcore_axis_name="s"),
        scratch_types=[
            pltpu.VMEM((NUM_EXPERTS, TPW), jnp.int32),
            pltpu.VMEM((TOP_K, TPW), jnp.float32),
            pltpu.VMEM((TOP_K, TPW), jnp.int32),
        ],
        interpret=interpret,
    )
    tw_t, ti_t = sc_call(keys_t)
    return probs, tw_t.T, ti_t.T


def kernel(hidden_states, weight):
    return _run(hidden_states, weight)


# speedup vs baseline: 1.0019x; 1.0019x over previous
import functools

import jax
import jax.numpy as jnp
from jax import lax
from jax.experimental import pallas as pl
from jax.experimental.pallas import tpu as pltpu
from jax.experimental.pallas import tpu_sc as plsc

NUM_EXPERTS = 64
TOP_K = 8
MODEL_DIM = 2048
T = 16384
BLOCK_T = 1024

NC = 2
NS = 16
NW = NC * NS
TPW = T // NW
GROUPS = TPW // 16

INT_MIN = -(2 ** 31)


def _tc_block(x_ref, w_ref, probs_ref, keys_ref):
    x = x_ref[...]
    w = w_ref[...]
    logits_t = lax.dot_general(
        w, x,
        dimension_numbers=(((1,), (1,)), ((), ())),
        preferred_element_type=jnp.float32,
    )
    m = jnp.max(logits_t, axis=0, keepdims=True)
    e = jnp.exp(logits_t - m)
    s = jnp.sum(e, axis=0, keepdims=True)
    probs_ref[...] = (e * (1.0 / s)).T
    iota_e = lax.broadcasted_iota(jnp.int32, e.shape, 0)
    keys_ref[...] = (lax.bitcast_convert_type(e, jnp.int32) & ~63) | (63 - iota_e)


def _sc_topk(keys_hbm, tw_hbm, ti_hbm, keys_v, tw_v, ti_v):
    wid = lax.axis_index("s") * NC + lax.axis_index("c")
    base = wid * TPW
    pltpu.sync_copy(keys_hbm.at[:, pl.ds(base, TPW)], keys_v)

    @plsc.parallel_loop(0, GROUPS, 1, unroll=2)
    def group(g):
        t = [jnp.full((16,), INT_MIN, jnp.int32) for _ in range(TOP_K)]
        for ex in range(NUM_EXPERTS):
            v = keys_v[ex, pl.ds(g * 16, 16)]
            c = [v > tj for tj in t]
            nt = [jnp.where(c[0], v, t[0])]
            for j in range(1, TOP_K):
                nt.append(jnp.where(c[j], jnp.where(c[j - 1], t[j - 1], v), t[j]))
            t = nt
        vals = [lax.bitcast_convert_type((tj & ~63) | 32, jnp.float32) for tj in t]
        ssum = vals[0]
        for vv in vals[1:]:
            ssum = ssum + vv
        inv = 1.0 / ssum
        for j in range(TOP_K):
            tw_v[j, pl.ds(g * 16, 16)] = vals[j] * inv
            ti_v[j, pl.ds(g * 16, 16)] = 63 - (t[j] & 63)
    pltpu.sync_copy(tw_v, tw_hbm.at[:, pl.ds(base, TPW)])
    pltpu.sync_copy(ti_v, ti_hbm.at[:, pl.ds(base, TPW)])


@functools.partial(jax.jit, static_argnames=("interpret",))
def _run(hidden_states, weight, interpret=False):
    x = hidden_states.reshape(-1, MODEL_DIM)
    grid = (T // BLOCK_T,)
    probs, keys_t = pl.pallas_call(
        _tc_block,
        grid=grid,
        in_specs=[
            pl.BlockSpec((BLOCK_T, MODEL_DIM), lambda i: (i, 0)),
            pl.BlockSpec((NUM_EXPERTS, MODEL_DIM), lambda i: (0, 0)),
        ],
        out_specs=[
            pl.BlockSpec((BLOCK_T, NUM_EXPERTS), lambda i: (i, 0)),
            pl.BlockSpec((NUM_EXPERTS, BLOCK_T), lambda i: (0, i)),
        ],
        out_shape=[
            jax.ShapeDtypeStruct((T, NUM_EXPERTS), jnp.float32),
            jax.ShapeDtypeStruct((NUM_EXPERTS, T), jnp.int32),
        ],
        interpret=interpret,
    )(x, weight)

    sc_call = pl.kernel(
        _sc_topk,
        out_type=[
            jax.ShapeDtypeStruct((TOP_K, T), jnp.float32),
            jax.ShapeDtypeStruct((TOP_K, T), jnp.int32),
        ],
        mesh=plsc.VectorSubcoreMesh(core_axis_name="c", subcore_axis_name="s"),
        scratch_types=[
            pltpu.VMEM((NUM_EXPERTS, TPW), jnp.int32),
            pltpu.VMEM((TOP_K, TPW), jnp.float32),
            pltpu.VMEM((TOP_K, TPW), jnp.int32),
        ],
        interpret=interpret,
    )
    tw_t, ti_t = sc_call(keys_t)
    return probs, tw_t.T, ti_t.T


def kernel(hidden_states, weight):
    return _run(hidden_states, weight)
